# baseline (speedup 1.0000x reference)
import functools
import jax
import jax.numpy as jnp
from jax import lax
from jax.experimental import pallas as pl
from jax.experimental.pallas import tpu as pltpu, tpu_sc as plsc

D_MODEL = 1024
SEQ_LEN = 2048
BATCH = 4
SCALE = 32.0

NUM_CORES = 2
NUM_SUBCORES = 16
NUM_WORKERS = NUM_CORES * NUM_SUBCORES
S_PER_W = SEQ_LEN // NUM_WORKERS
CHUNK = 32
LANES = 16

_mesh = plsc.VectorSubcoreMesh(core_axis_name="c", subcore_axis_name="s")


@functools.partial(
    pl.kernel,
    mesh=_mesh,
    out_type=jax.ShapeDtypeStruct((BATCH * SEQ_LEN, D_MODEL), jnp.float32),
    scratch_types=[
        pltpu.VMEM((CHUNK,), jnp.int32),
        pltpu.VMEM((CHUNK, D_MODEL), jnp.float32),
        pltpu.VMEM((CHUNK, D_MODEL), jnp.float32),
        pltpu.SemaphoreType.DMA,
    ],
)
def _embed(x_hbm, table_hbm, pe_hbm, out_hbm, idx_v, pe_v, rows_v, sem):
    wid = lax.axis_index("s") * NUM_CORES + lax.axis_index("c")
    s_base = wid * S_PER_W

    for half in range(S_PER_W // CHUNK):
        s0 = s_base + half * CHUNK
        pltpu.sync_copy(pe_hbm.at[pl.ds(s0, CHUNK)], pe_v)
        for b in range(BATCH):
            flat = b * SEQ_LEN + s0
            pltpu.sync_copy(x_hbm.at[pl.ds(flat, CHUNK)], idx_v)
            pltpu.async_copy(table_hbm.at[idx_v], rows_v, sem).wait()

            def row_body(r, _):
                def lane_body(j, _):
                    c = j * LANES
                    v = rows_v[r, pl.ds(c, LANES)] * SCALE + pe_v[r, pl.ds(c, LANES)]
                    rows_v[r, pl.ds(c, LANES)] = v
                    return 0

                return lax.fori_loop(0, D_MODEL // LANES, lane_body, 0)

            lax.fori_loop(0, CHUNK, row_body, 0)
            pltpu.sync_copy(rows_v, out_hbm.at[pl.ds(flat, CHUNK)])


def kernel(x, token_table, pe):
    x_flat = x.reshape(-1).astype(jnp.int32)
    pe_flat = pe.reshape(SEQ_LEN, D_MODEL)
    out = _embed(x_flat, token_table, pe_flat)
    return out.reshape(BATCH, SEQ_LEN, D_MODEL)


# speedup vs baseline: 2.0974x; 2.0974x over previous
import functools
import jax
import jax.numpy as jnp
from jax import lax
from jax.experimental import pallas as pl
from jax.experimental.pallas import tpu as pltpu, tpu_sc as plsc

D_MODEL = 1024
SEQ_LEN = 2048
BATCH = 4
SCALE = 32.0

NUM_CORES = 2
NUM_SUBCORES = 16
NUM_WORKERS = NUM_CORES * NUM_SUBCORES
S_PER_W = SEQ_LEN // NUM_WORKERS
HALVES = 2
S_PER_HALF = S_PER_W // HALVES
CHUNK_P = 8
CHUNKS = S_PER_HALF // CHUNK_P
ROWS = BATCH * CHUNK_P
LANES = 16
NSLICE = D_MODEL // LANES

_mesh = plsc.VectorSubcoreMesh(core_axis_name="c", subcore_axis_name="s")


@functools.partial(
    pl.kernel,
    mesh=_mesh,
    out_type=jax.ShapeDtypeStruct((BATCH * SEQ_LEN, D_MODEL), jnp.float32),
    scratch_types=[
        pltpu.VMEM((BATCH * S_PER_W,), jnp.int32),
        pltpu.VMEM((S_PER_HALF, D_MODEL), jnp.float32),
        pltpu.VMEM((ROWS, D_MODEL), jnp.float32),
        pltpu.SemaphoreType.DMA,
    ],
)
def _embed(idx_hbm, table_hbm, pe_hbm, out_hbm, idx_v, pe_v, rows_v, sem):
    wid = lax.axis_index("s") * NUM_CORES + lax.axis_index("c")
    s_base = wid * S_PER_W

    pltpu.sync_copy(idx_hbm.at[pl.ds(wid * BATCH * S_PER_W, BATCH * S_PER_W)], idx_v)

    for h in range(HALVES):
        s_half = s_base + h * S_PER_HALF
        pltpu.sync_copy(pe_hbm.at[pl.ds(s_half, S_PER_HALF)], pe_v)
        for k in range(CHUNKS):
            idx_off = (h * CHUNKS + k) * ROWS
            pltpu.async_copy(
                table_hbm.at[idx_v.at[pl.ds(idx_off, ROWS)]], rows_v, sem
            ).wait()

            pe_row0 = k * CHUNK_P

            @plsc.parallel_loop(0, CHUNK_P)
            def _p_loop(p):
                @plsc.parallel_loop(0, NSLICE, unroll=4)
                def _j_loop(j):
                    c = j * LANES
                    pe_vec = pe_v[pe_row0 + p, pl.ds(c, LANES)]
                    for b in range(BATCH):
                        r = b * CHUNK_P + p
                        rows_v[r, pl.ds(c, LANES)] = (
                            rows_v[r, pl.ds(c, LANES)] * SCALE + pe_vec
                        )

            for b in range(BATCH):
                pltpu.sync_copy(
                    rows_v.at[pl.ds(b * CHUNK_P, CHUNK_P)],
                    out_hbm.at[pl.ds(b * SEQ_LEN + s_half + k * CHUNK_P, CHUNK_P)],
                )


def kernel(x, token_table, pe):
    xr = x.reshape(BATCH, NUM_WORKERS, HALVES * CHUNKS, CHUNK_P)
    idx_flat = xr.transpose(1, 2, 0, 3).reshape(-1).astype(jnp.int32)
    pe_flat = pe.reshape(SEQ_LEN, D_MODEL)
    out = _embed(idx_flat, token_table, pe_flat)
    return out.reshape(BATCH, SEQ_LEN, D_MODEL)


# speedup vs baseline: 2.8363x; 1.3523x over previous
import functools
import jax
import jax.numpy as jnp
from jax import lax
from jax.experimental import pallas as pl
from jax.experimental.pallas import tpu as pltpu, tpu_sc as plsc

D_MODEL = 1024
SEQ_LEN = 2048
BATCH = 4
SCALE = 32.0

NUM_CORES = 2
NUM_SUBCORES = 16
NUM_WORKERS = NUM_CORES * NUM_SUBCORES
S_PER_W = SEQ_LEN // NUM_WORKERS
CHUNK_P = 8
CHUNKS = S_PER_W // CHUNK_P
ROWS = BATCH * CHUNK_P
Q_P = 16
QUARTERS = S_PER_W // Q_P
LANES = 16
NSLICE = D_MODEL // LANES

_mesh = plsc.VectorSubcoreMesh(core_axis_name="c", subcore_axis_name="s")


@functools.partial(
    pl.kernel,
    mesh=_mesh,
    out_type=jax.ShapeDtypeStruct((BATCH * SEQ_LEN, D_MODEL), jnp.float32),
    scratch_types=[
        pltpu.VMEM((BATCH * S_PER_W,), jnp.int32),
        pltpu.VMEM((2, Q_P, D_MODEL), jnp.float32),
        pltpu.VMEM((2, ROWS, D_MODEL), jnp.float32),
        pltpu.SemaphoreType.DMA,
        pltpu.SemaphoreType.DMA,
        pltpu.SemaphoreType.DMA,
    ],
)
def _embed(idx_hbm, table_hbm, pe_hbm, out_hbm, idx_v, pe_v, rows_v, gsem, ssem, psem):
    wid = lax.axis_index("s") * NUM_CORES + lax.axis_index("c")
    s_base = wid * S_PER_W

    pltpu.sync_copy(idx_hbm.at[pl.ds(wid * BATCH * S_PER_W, BATCH * S_PER_W)], idx_v)

    def gather(g):
        return pltpu.async_copy(
            table_hbm.at[idx_v.at[pl.ds(g * ROWS, ROWS)]], rows_v.at[g % 2], gsem
        )

    def pe_load(q):
        return pltpu.async_copy(
            pe_hbm.at[pl.ds(s_base + q * Q_P, Q_P)], pe_v.at[q % 2], psem
        )

    pe_handles = {0: pe_load(0)}
    g_handles = {0: gather(0)}
    store_handles = {}
    pe_handles[0].wait()

    for g in range(CHUNKS):
        q = g * CHUNK_P // Q_P
        buf = g % 2
        if g + 1 < CHUNKS:
            if g - 1 >= 0:
                for h in store_handles.pop(g - 1):
                    h.wait()
            g_handles[g + 1] = gather(g + 1)
        if g % 2 == 0 and q + 1 < QUARTERS:
            pe_handles[q + 1] = pe_load(q + 1)
        g_handles.pop(g).wait()
        if g % 2 == 0 and g > 0:
            pe_handles.pop(q).wait()

        pe_row0 = (g % 2) * CHUNK_P
        pebuf = q % 2

        @plsc.parallel_loop(0, CHUNK_P)
        def _p_loop(p):
            @plsc.parallel_loop(0, NSLICE, unroll=4)
            def _j_loop(j):
                c = j * LANES
                pe_vec = pe_v[pebuf, pe_row0 + p, pl.ds(c, LANES)]
                for b in range(BATCH):
                    r = b * CHUNK_P + p
                    rows_v[buf, r, pl.ds(c, LANES)] = (
                        rows_v[buf, r, pl.ds(c, LANES)] * SCALE + pe_vec
                    )

        store_handles[g] = [
            pltpu.async_copy(
                rows_v.at[buf, pl.ds(b * CHUNK_P, CHUNK_P)],
                out_hbm.at[pl.ds(b * SEQ_LEN + s_base + g * CHUNK_P, CHUNK_P)],
                ssem,
            )
            for b in range(BATCH)
        ]

    for g in sorted(store_handles):
        for h in store_handles[g]:
            h.wait()


def kernel(x, token_table, pe):
    xr = x.reshape(BATCH, NUM_WORKERS, CHUNKS, CHUNK_P)
    idx_flat = xr.transpose(1, 2, 0, 3).reshape(-1).astype(jnp.int32)
    pe_flat = pe.reshape(SEQ_LEN, D_MODEL)
    out = _embed(idx_flat, token_table, pe_flat)
    return out.reshape(BATCH, SEQ_LEN, D_MODEL)
